# baseline (speedup 1.0000x reference)
import functools

import jax
import jax.numpy as jnp
from jax import lax
from jax.experimental import pallas as pl
from jax.experimental.pallas import tpu as pltpu
from jax.experimental.pallas import tpu_sc as plsc

H = 128
CHUNK = 80
NW = 32


def _sc_gather(h, c, src, half, n, cpw, e):
    mesh = plsc.VectorSubcoreMesh(core_axis_name="c", subcore_axis_name="s")
    kpw = cpw * CHUNK
    num_chunks = n // CHUNK

    @functools.partial(
        pl.kernel,
        mesh=mesh,
        out_type=[jax.ShapeDtypeStruct((n, H), jnp.int32),
                  jax.ShapeDtypeStruct((n, H), jnp.int32)],
        scratch_types=[
            pltpu.VMEM((2 * kpw,), jnp.int32),
            pltpu.VMEM((kpw,), jnp.int32),
            pltpu.VMEM((kpw,), jnp.int32),
            pltpu.VMEM((2, CHUNK, H), jnp.float32),
            pltpu.VMEM((2, CHUNK, H), jnp.float32),
            pltpu.VMEM((2, CHUNK, H), jnp.float32),
            pltpu.VMEM((2, CHUNK, H), jnp.float32),
            pltpu.VMEM((CHUNK, H), jnp.int32),
            pltpu.VMEM((CHUNK, H), jnp.int32),
            pltpu.SemaphoreType.DMA,
            pltpu.SemaphoreType.DMA,
        ],
    )
    def k(h_hbm, c_hbm, src_hbm, hg_hbm, cg_hbm,
          src_v, idx_e, idx_o, he, ho, ce, co, hpk, cpk, g0, g1):
        wid = lax.axis_index("s") * 2 + lax.axis_index("c")
        start = cpw * wid
        nproc = jnp.maximum(0, jnp.minimum(cpw, num_chunks - start))
        off_raw = 2 * (half * n + start * CHUNK)
        off = pl.multiple_of(jnp.minimum(off_raw, e - 2 * kpw), 8)
        d = off_raw - off
        pltpu.sync_copy(src_hbm.at[pl.ds(off, 2 * kpw)], src_v)

        lane = lax.iota(jnp.int32, 16)
        perm_e = (lane % 8) * 2
        in_lo = lane < 8
        dnums = lax.GatherDimensionNumbers(
            offset_dims=(), collapsed_slice_dims=(0,), start_index_map=(0,))

        def _dg(a, idx):
            return lax.gather(a, idx[:, None], dnums, (1,),
                              mode=lax.GatherScatterMode.PROMISE_IN_BOUNDS)

        def deint(v, carry):
            a = src_v[pl.ds(d + 32 * v, 16)]
            b = src_v[pl.ds(d + 32 * v + 16, 16)]
            idx_e[pl.ds(v * 16, 16)] = jnp.where(
                in_lo, _dg(a, perm_e), _dg(b, perm_e))
            idx_o[pl.ds(v * 16, 16)] = jnp.where(
                in_lo, _dg(a, perm_e + 1), _dg(b, perm_e + 1))
            return carry

        lax.fori_loop(0, nproc * (CHUNK // 16), deint, 0)
        gsem = (g0, g1)

        def ie(j):
            return idx_e.at[pl.ds(j * CHUNK, CHUNK)]

        def io(j):
            return idx_o.at[pl.ds(j * CHUNK, CHUNK)]

        def issue(j, b):
            pltpu.async_copy(h_hbm.at[ie(j)], he.at[b], gsem[b])
            pltpu.async_copy(h_hbm.at[io(j)], ho.at[b], gsem[b])
            pltpu.async_copy(c_hbm.at[ie(j)], ce.at[b], gsem[b])
            pltpu.async_copy(c_hbm.at[io(j)], co.at[b], gsem[b])

        himask = jnp.int32(-65536)

        def conv(src_e, src_o, b, dst):
            def rowconv(r, carry):
                for g in range(H // 32):
                    for src, base in ((src_e, 0), (src_o, H // 2)):
                        a = lax.bitcast_convert_type(
                            src[b, r, pl.ds(g * 32, 16)], jnp.int32)
                        bb = lax.bitcast_convert_type(
                            src[b, r, pl.ds(g * 32 + 16, 16)], jnp.int32)
                        dst[r, pl.ds(base + g * 16, 16)] = (
                            lax.shift_right_logical(a, 16) | (bb & himask))
                return carry

            lax.fori_loop(0, CHUNK, rowconv, 0)

        def drain_store(j, b):
            pltpu.make_async_copy(h_hbm.at[ie(j)], he.at[b], gsem[b]).wait()
            pltpu.make_async_copy(h_hbm.at[io(j)], ho.at[b], gsem[b]).wait()
            pltpu.make_async_copy(c_hbm.at[ie(j)], ce.at[b], gsem[b]).wait()
            pltpu.make_async_copy(c_hbm.at[io(j)], co.at[b], gsem[b]).wait()
            conv(he, ho, b, hpk)
            conv(ce, co, b, cpk)
            rows = pl.ds((start + j) * CHUNK, CHUNK)
            pltpu.sync_copy(hpk, hg_hbm.at[rows])
            pltpu.sync_copy(cpk, cg_hbm.at[rows])

        @pl.when(nproc > 0)
        def _():
            issue(0, 0)

        def body(t, carry):
            for phase in (0, 1):
                j = 2 * t + phase
                jn = j + 1

                @pl.when(jn < nproc)
                def _():
                    issue(jn, 1 - phase)

                @pl.when(j < nproc)
                def _():
                    drain_store(j, phase)
            return carry

        lax.fori_loop(0, (cpw + 1) // 2, body, 0, unroll=False)

    return k(h, c, src)


def _unpack_halves(words):
    lo = lax.bitcast_convert_type(lax.shift_left(words, 16), jnp.float32)
    hi = lax.bitcast_convert_type(words & jnp.int32(-65536), jnp.float32)
    return lo, hi


def _lane_take(arr, idx):
    return jnp.take_along_axis(
        arr, jnp.broadcast_to(idx[None, :], (arr.shape[0], idx.shape[0])),
        axis=1)


def _tc_cell_body(x_ref, hcat_ref, mc_ref, leaf_ref, w_ref, b_ref, ul_ref,
                  uh_ref, bu_ref, ufl_ref, ufh_ref, bf_ref, *refs):
    h_out, c_out = refs[-2], refs[-1]
    h_lo, h_hi = _unpack_halves(hcat_ref[...])
    mc_lo, mc_hi = _unpack_halves(mc_ref[...])
    f = jax.nn.sigmoid(
        jnp.dot(h_lo, ufl_ref[...], preferred_element_type=jnp.float32)
        + jnp.dot(h_hi, ufh_ref[...], preferred_element_type=jnp.float32)
        + bf_ref[...])
    w = lax.iota(jnp.int32, H)
    m = w % 64
    gfeat = (m // 16) * 32 + (m % 16)
    fl, fr = f[:, :H], f[:, H:]
    is_left = (w < 64)[None, :]
    f_lo = jnp.where(is_left, _lane_take(fl, gfeat), _lane_take(fr, gfeat))
    f_hi = jnp.where(is_left, _lane_take(fl, gfeat + 16),
                     _lane_take(fr, gfeat + 16))
    fc_lo = f_lo * mc_lo
    fc_hi = f_hi * mc_hi
    cf_a = fc_lo[:, :H // 2] + fc_lo[:, H // 2:]
    cf_b = fc_hi[:, :H // 2] + fc_hi[:, H // 2:]
    cf_cat = jnp.concatenate([cf_a, cf_b], axis=1)
    k = lax.iota(jnp.int32, H)
    j = k % 32
    cfp = jnp.where(j < 16, (k // 32) * 16 + j,
                    H // 2 + (k // 32) * 16 + (j - 16))
    cf = _lane_take(cf_cat, cfp)
    leaf = leaf_ref[0, 0, :].reshape(-1, 1)
    xw = jnp.dot(x_ref[...], w_ref[...],
                 preferred_element_type=jnp.float32) + b_ref[...]
    hu = (jnp.dot(h_lo, ul_ref[...], preferred_element_type=jnp.float32)
          + jnp.dot(h_hi, uh_ref[...], preferred_element_type=jnp.float32)
          + bu_ref[...])
    iou = leaf * xw + (1.0 - leaf) * hu
    gi = jax.nn.sigmoid(iou[:, :H])
    go = jax.nn.sigmoid(iou[:, H:2 * H])
    gu = jnp.tanh(iou[:, 2 * H:])
    c_new = gi * gu + cf
    h_out[...] = go * jnp.tanh(c_new)
    c_out[...] = c_new


def _tc_cell(x, hcat, mc, leaf, w_iou, b_iou, u_iou_lo, u_iou_hi, bu_iou,
             u_f_lo, u_f_hi, b_f, blk, base_blk, prev=None):
    n = x.shape[0]
    nb = hcat.shape[0] // blk
    row = lambda i: (i, 0)
    rowb = lambda i: (i + base_blk, 0)
    rep = lambda i: (0, 0)
    in_specs = [
        pl.BlockSpec((blk, H), rowb),
        pl.BlockSpec((blk, H), row),
        pl.BlockSpec((blk, H), row),
        pl.BlockSpec((1, 1, blk), lambda i: (i + base_blk, 0, 0)),
        pl.BlockSpec((H, 3 * H), rep),
        pl.BlockSpec((1, 3 * H), rep),
        pl.BlockSpec((H, 3 * H), rep),
        pl.BlockSpec((H, 3 * H), rep),
        pl.BlockSpec((1, 3 * H), rep),
        pl.BlockSpec((H, 2 * H), rep),
        pl.BlockSpec((H, 2 * H), rep),
        pl.BlockSpec((1, 2 * H), rep),
    ]
    args = [x, hcat, mc, leaf, w_iou, b_iou, u_iou_lo, u_iou_hi, bu_iou,
            u_f_lo, u_f_hi, b_f]
    aliases = {}
    if prev is not None:
        in_specs += [pl.BlockSpec(memory_space=pl.ANY)] * 2
        args += [prev[0], prev[1]]
        aliases = {12: 0, 13: 1}
    return pl.pallas_call(
        _tc_cell_body,
        grid=(nb,),
        in_specs=in_specs,
        out_specs=[pl.BlockSpec((blk, H), rowb), pl.BlockSpec((blk, H), rowb)],
        out_shape=[jax.ShapeDtypeStruct((n, H), jnp.float32)] * 2,
        input_output_aliases=aliases,
    )(*args)


def kernel(x, h, c, is_leaf, edge_index, w_iou, b_iou, u_iou, bu_iou, u_f, b_f):
    n = x.shape[0]
    e = edge_index.shape[1]
    nh = n // 2
    cpw = -(-(nh // CHUNK) // NW)
    src = edge_index[0]
    halves = [_sc_gather(h, c, src, p, nh, cpw, e) for p in range(2)]
    blk = 2000
    leaf = is_leaf.astype(jnp.float32).reshape(n // blk, 1, blk)
    w2 = jnp.arange(H, dtype=jnp.int32)
    lo_orig = (w2 // 64) * H + ((w2 % 64) // 16) * 32 + (w2 % 16)
    u_iou_lo = u_iou[lo_orig]
    u_iou_hi = u_iou[lo_orig + 16]
    u_f_lo = u_f[lo_orig]
    u_f_hi = u_f[lo_orig + 16]
    b_iou2 = b_iou.reshape(1, -1)
    bu_iou2 = bu_iou.reshape(1, -1)
    b_f2 = b_f.reshape(1, -1)
    wargs = (w_iou, b_iou2, u_iou_lo, u_iou_hi, bu_iou2,
             u_f_lo, u_f_hi, b_f2)
    h1, c1 = _tc_cell(x, halves[0][0], halves[0][1], leaf, *wargs,
                      blk, 0)
    h2, c2 = _tc_cell(x, halves[1][0], halves[1][1], leaf, *wargs,
                      blk, nh // blk, prev=(h1, c1))
    return h2, c2


# speedup vs baseline: 2.0937x; 2.0937x over previous
import functools

import jax
import jax.numpy as jnp
from jax import lax
from jax.experimental import pallas as pl
from jax.experimental.pallas import tpu as pltpu
from jax.experimental.pallas import tpu_sc as plsc

H = 128
CHUNK = 80
NW = 32


def _sc_gather(h, c, src, half, n, cpw, e):
    mesh = plsc.VectorSubcoreMesh(core_axis_name="c", subcore_axis_name="s")
    kpw = cpw * CHUNK
    num_chunks = n // CHUNK

    @functools.partial(
        pl.kernel,
        mesh=mesh,
        out_type=[jax.ShapeDtypeStruct((n, H), jnp.int32),
                  jax.ShapeDtypeStruct((n, H), jnp.int32)],
        scratch_types=[
            pltpu.VMEM((2 * kpw,), jnp.int32),
            pltpu.VMEM((kpw,), jnp.int32),
            pltpu.VMEM((kpw,), jnp.int32),
            pltpu.VMEM((2, CHUNK, H), jnp.float32),
            pltpu.VMEM((2, CHUNK, H), jnp.float32),
            pltpu.VMEM((2, CHUNK, H), jnp.float32),
            pltpu.VMEM((2, CHUNK, H), jnp.float32),
            pltpu.VMEM((CHUNK, H), jnp.int32),
            pltpu.VMEM((CHUNK, H), jnp.int32),
            pltpu.SemaphoreType.DMA,
            pltpu.SemaphoreType.DMA,
        ],
    )
    def k(h_hbm, c_hbm, src_hbm, hg_hbm, cg_hbm,
          src_v, idx_e, idx_o, he, ho, ce, co, hpk, cpk, g0, g1):
        wid = lax.axis_index("s") * 2 + lax.axis_index("c")
        start = cpw * wid
        nproc = jnp.maximum(0, jnp.minimum(cpw, num_chunks - start))
        off_raw = 2 * (half * n + start * CHUNK)
        off = pl.multiple_of(jnp.minimum(off_raw, e - 2 * kpw), 8)
        d = off_raw - off
        pltpu.sync_copy(src_hbm.at[pl.ds(off, 2 * kpw)], src_v)

        lane = lax.iota(jnp.int32, 16)
        perm_e = (lane % 8) * 2
        in_lo = lane < 8
        dnums = lax.GatherDimensionNumbers(
            offset_dims=(), collapsed_slice_dims=(0,), start_index_map=(0,))

        def _dg(a, idx):
            return lax.gather(a, idx[:, None], dnums, (1,),
                              mode=lax.GatherScatterMode.PROMISE_IN_BOUNDS)

        def deint(v, carry):
            a = src_v[pl.ds(d + 32 * v, 16)]
            b = src_v[pl.ds(d + 32 * v + 16, 16)]
            idx_e[pl.ds(v * 16, 16)] = jnp.where(
                in_lo, _dg(a, perm_e), _dg(b, perm_e))
            idx_o[pl.ds(v * 16, 16)] = jnp.where(
                in_lo, _dg(a, perm_e + 1), _dg(b, perm_e + 1))
            return carry

        lax.fori_loop(0, nproc * (CHUNK // 16), deint, 0)
        gsem = (g0, g1)

        def ie(j):
            return idx_e.at[pl.ds(j * CHUNK, CHUNK)]

        def io(j):
            return idx_o.at[pl.ds(j * CHUNK, CHUNK)]

        def issue(j, b):
            pltpu.async_copy(h_hbm.at[ie(j)], he.at[b], gsem[b])
            pltpu.async_copy(h_hbm.at[io(j)], ho.at[b], gsem[b])
            pltpu.async_copy(c_hbm.at[ie(j)], ce.at[b], gsem[b])
            pltpu.async_copy(c_hbm.at[io(j)], co.at[b], gsem[b])

        himask = jnp.int32(-65536)

        def conv(src_e, src_o, b, dst):
            def rowconv(r, carry):
                for g in range(H // 16):
                    a = lax.bitcast_convert_type(
                        src_e[b, r, pl.ds(g * 16, 16)], jnp.int32)
                    bb = lax.bitcast_convert_type(
                        src_o[b, r, pl.ds(g * 16, 16)], jnp.int32)
                    dst[r, pl.ds(g * 16, 16)] = (
                        lax.shift_right_logical(a, 16) | (bb & himask))
                return carry

            lax.fori_loop(0, CHUNK, rowconv, 0)

        def drain_store(j, b):
            pltpu.make_async_copy(h_hbm.at[ie(j)], he.at[b], gsem[b]).wait()
            pltpu.make_async_copy(h_hbm.at[io(j)], ho.at[b], gsem[b]).wait()
            pltpu.make_async_copy(c_hbm.at[ie(j)], ce.at[b], gsem[b]).wait()
            pltpu.make_async_copy(c_hbm.at[io(j)], co.at[b], gsem[b]).wait()
            conv(he, ho, b, hpk)
            conv(ce, co, b, cpk)
            rows = pl.ds((start + j) * CHUNK, CHUNK)
            pltpu.sync_copy(hpk, hg_hbm.at[rows])
            pltpu.sync_copy(cpk, cg_hbm.at[rows])

        @pl.when(nproc > 0)
        def _():
            issue(0, 0)

        def body(t, carry):
            for phase in (0, 1):
                j = 2 * t + phase
                jn = j + 1

                @pl.when(jn < nproc)
                def _():
                    issue(jn, 1 - phase)

                @pl.when(j < nproc)
                def _():
                    drain_store(j, phase)
            return carry

        lax.fori_loop(0, (cpw + 1) // 2, body, 0, unroll=False)

    return k(h, c, src)


def _unpack_halves(words):
    lo = lax.bitcast_convert_type(lax.shift_left(words, 16), jnp.float32)
    hi = lax.bitcast_convert_type(words & jnp.int32(-65536), jnp.float32)
    return lo, hi


def _lane_take(arr, idx):
    return jnp.take_along_axis(
        arr, jnp.broadcast_to(idx[None, :], (arr.shape[0], idx.shape[0])),
        axis=1)


def _tc_cell_body(x_ref, hcat_ref, mc_ref, leaf_ref, w_ref, b_ref, ul_ref,
                  uh_ref, bu_ref, ufl_ref, ufh_ref, bf_ref, *refs):
    h_out, c_out = refs[-2], refs[-1]
    h_lo, h_hi = _unpack_halves(hcat_ref[...])
    mc_lo, mc_hi = _unpack_halves(mc_ref[...])
    f = jax.nn.sigmoid(
        jnp.dot(h_lo, ufl_ref[...], preferred_element_type=jnp.float32)
        + jnp.dot(h_hi, ufh_ref[...], preferred_element_type=jnp.float32)
        + bf_ref[...])
    cf = f[:, :H] * mc_lo + f[:, H:] * mc_hi
    leaf = leaf_ref[0, 0, :].reshape(-1, 1)
    xw = jnp.dot(x_ref[...], w_ref[...],
                 preferred_element_type=jnp.float32) + b_ref[...]
    hu = (jnp.dot(h_lo, ul_ref[...], preferred_element_type=jnp.float32)
          + jnp.dot(h_hi, uh_ref[...], preferred_element_type=jnp.float32)
          + bu_ref[...])
    iou = leaf * xw + (1.0 - leaf) * hu
    gi = jax.nn.sigmoid(iou[:, :H])
    go = jax.nn.sigmoid(iou[:, H:2 * H])
    gu = jnp.tanh(iou[:, 2 * H:])
    c_new = gi * gu + cf
    h_out[...] = go * jnp.tanh(c_new)
    c_out[...] = c_new


def _tc_cell(x, hcat, mc, leaf, w_iou, b_iou, u_iou_lo, u_iou_hi, bu_iou,
             u_f_lo, u_f_hi, b_f, blk, base_blk, prev=None):
    n = x.shape[0]
    nb = hcat.shape[0] // blk
    row = lambda i: (i, 0)
    rowb = lambda i: (i + base_blk, 0)
    rep = lambda i: (0, 0)
    in_specs = [
        pl.BlockSpec((blk, H), rowb),
        pl.BlockSpec((blk, H), row),
        pl.BlockSpec((blk, H), row),
        pl.BlockSpec((1, 1, blk), lambda i: (i + base_blk, 0, 0)),
        pl.BlockSpec((H, 3 * H), rep),
        pl.BlockSpec((1, 3 * H), rep),
        pl.BlockSpec((H, 3 * H), rep),
        pl.BlockSpec((H, 3 * H), rep),
        pl.BlockSpec((1, 3 * H), rep),
        pl.BlockSpec((H, 2 * H), rep),
        pl.BlockSpec((H, 2 * H), rep),
        pl.BlockSpec((1, 2 * H), rep),
    ]
    args = [x, hcat, mc, leaf, w_iou, b_iou, u_iou_lo, u_iou_hi, bu_iou,
            u_f_lo, u_f_hi, b_f]
    aliases = {}
    if prev is not None:
        in_specs += [pl.BlockSpec(memory_space=pl.ANY)] * 2
        args += [prev[0], prev[1]]
        aliases = {12: 0, 13: 1}
    return pl.pallas_call(
        _tc_cell_body,
        grid=(nb,),
        in_specs=in_specs,
        out_specs=[pl.BlockSpec((blk, H), rowb), pl.BlockSpec((blk, H), rowb)],
        out_shape=[jax.ShapeDtypeStruct((n, H), jnp.float32)] * 2,
        input_output_aliases=aliases,
    )(*args)


def kernel(x, h, c, is_leaf, edge_index, w_iou, b_iou, u_iou, bu_iou, u_f, b_f):
    n = x.shape[0]
    e = edge_index.shape[1]
    nh = n // 2
    cpw = -(-(nh // CHUNK) // NW)
    src = edge_index[0]
    halves = [_sc_gather(h, c, src, p, nh, cpw, e) for p in range(2)]
    blk = 2000
    leaf = is_leaf.astype(jnp.float32).reshape(n // blk, 1, blk)
    u_iou_lo = u_iou[:H]
    u_iou_hi = u_iou[H:]
    u_f_lo = u_f[:H]
    u_f_hi = u_f[H:]
    b_iou2 = b_iou.reshape(1, -1)
    bu_iou2 = bu_iou.reshape(1, -1)
    b_f2 = b_f.reshape(1, -1)
    wargs = (w_iou, b_iou2, u_iou_lo, u_iou_hi, bu_iou2,
             u_f_lo, u_f_hi, b_f2)
    h1, c1 = _tc_cell(x, halves[0][0], halves[0][1], leaf, *wargs,
                      blk, 0)
    h2, c2 = _tc_cell(x, halves[1][0], halves[1][1], leaf, *wargs,
                      blk, nh // blk, prev=(h1, c1))
    return h2, c2
